# baseline (speedup 1.0000x reference)
import functools

import jax
import jax.numpy as jnp
from jax import lax
from jax.experimental import pallas as pl
from jax.experimental.pallas import tpu as pltpu
from jax.experimental.pallas import tpu_sc as plsc

_N = 10000
_E = 320000
_D = 128
_DH = _D // 2
_NG = 64
_L = 3
_EPS = 1e-5

_NC, _NS = 2, 16
_NW = _NC * _NS
_CH = 112
_IB = 24
_E2 = 2 * _E
_EPW = -(-_E2 // (_NW * _CH * _IB)) * (_CH * _IB)
_CPW = _EPW // _CH
_NBLK = _CPW // _IB
_EPAD = _NW * _EPW
_NPADROWS = 16
_NPAD = _N + _NPADROWS
_RPT = 640
_RPT_LAST = _N - _RPT * (_NS - 1)


_mesh = plsc.VectorSubcoreMesh(core_axis_name="c", subcore_axis_name="s",
                               num_cores=_NC, num_subcores=_NS)
_f32 = jnp.float32


@functools.partial(
    pl.kernel,
    mesh=_mesh,
    out_type=jax.ShapeDtypeStruct((_NC, _N), _f32),
    scratch_types=[
        pltpu.VMEM((_CPW, _CH), jnp.int32),
        pltpu.VMEM((_CPW, _CH), _f32),
        pltpu.VMEM_SHARED((_N,), _f32),
    ],
)
def _deg_kernel(sidx_hbm, ew_hbm, zeros1_hbm, out_hbm, idx_v, ew_v, dacc):
    cid = lax.axis_index("c")
    sid = lax.axis_index("s")
    wid = cid * _NS + sid
    pltpu.sync_copy(sidx_hbm.at[wid], idx_v)
    pltpu.sync_copy(ew_hbm.at[wid], ew_v)

    @pl.when(sid == 0)
    def _():
        pltpu.sync_copy(zeros1_hbm, dacc)

    plsc.subcore_barrier()

    def body(j, c):
        pltpu.sync_copy(ew_v.at[j], dacc.at[idx_v.at[j]], add=True)
        return c

    lax.fori_loop(0, _CPW, body, 0)
    plsc.subcore_barrier()

    @pl.when(sid == 0)
    def _():
        pltpu.sync_copy(dacc, out_hbm.at[cid])


@functools.partial(
    pl.kernel,
    mesh=_mesh,
    out_type=jax.ShapeDtypeStruct((_NC, _N, _D), _f32),
    scratch_types=[
        pltpu.VMEM((_IB, _CH), jnp.int32),
        pltpu.VMEM((_IB, _CH), jnp.int32),
        pltpu.VMEM((_CH, _D), _f32),
        pltpu.VMEM((_CH, _D), _f32),
        pltpu.VMEM((_CH, _D), _f32),
        pltpu.VMEM_SHARED((_N, _D), _f32),
        pltpu.SemaphoreType.DMA,
        pltpu.SemaphoreType.DMA,
        pltpu.SemaphoreType.DMA,
        pltpu.SemaphoreType.DMA,
        pltpu.SemaphoreType.DMA,
        pltpu.SemaphoreType.DMA,
    ],
)
def _agg_kernel(h2_hbm, gidx_hbm, sidx_hbm, zeros2_hbm, out_hbm,
                gidx_v, sidx_v, r0, r1, r2, acc, g0, g1, g2, s0, s1, s2):
    cid = lax.axis_index("c")
    sid = lax.axis_index("s")
    wid = cid * _NS + sid
    roff = pl.multiple_of(sid * _RPT, 8)

    @pl.when(sid < _NS - 1)
    def _():
        pltpu.sync_copy(zeros2_hbm.at[pl.ds(roff, _RPT)],
                        acc.at[pl.ds(roff, _RPT)])

    @pl.when(sid == _NS - 1)
    def _():
        pltpu.sync_copy(zeros2_hbm.at[pl.ds(roff, _RPT_LAST)],
                        acc.at[pl.ds(roff, _RPT_LAST)])

    plsc.subcore_barrier()

    rows = (r0, r1, r2)
    gsem = (g0, g1, g2)
    ssem = (s0, s1, s2)

    def blk_body(blk, c):
        boff = pl.multiple_of(blk * _IB, _IB)
        pltpu.sync_copy(gidx_hbm.at[wid, pl.ds(boff, _IB)], gidx_v)
        pltpu.sync_copy(sidx_hbm.at[wid, pl.ds(boff, _IB)], sidx_v)
        pltpu.async_copy(h2_hbm.at[gidx_v.at[0]], rows[0], gsem[0])
        pltpu.async_copy(h2_hbm.at[gidx_v.at[1]], rows[1], gsem[1])
        for j in range(_IB):
            b = j % 3
            pltpu.make_async_copy(h2_hbm.at[gidx_v.at[j]], rows[b],
                                  gsem[b]).wait()
            pltpu.async_copy(rows[b], acc.at[sidx_v.at[j]], ssem[b],
                             add=True)
            if j + 2 < _IB:
                b2 = (j + 2) % 3
                if j >= 1:
                    pltpu.make_async_copy(rows[b2], acc.at[sidx_v.at[j - 1]],
                                          ssem[b2]).wait()
                pltpu.async_copy(h2_hbm.at[gidx_v.at[j + 2]], rows[b2],
                                 gsem[b2])
        for j in range(_IB - 3, _IB):
            b = j % 3
            pltpu.make_async_copy(rows[b], acc.at[sidx_v.at[j]],
                                  ssem[b]).wait()
        return c

    lax.fori_loop(0, _NBLK, blk_body, 0)
    plsc.subcore_barrier()

    @pl.when(sid < _NS - 1)
    def _():
        pltpu.sync_copy(acc.at[pl.ds(roff, _RPT)],
                        out_hbm.at[cid, pl.ds(roff, _RPT)])

    @pl.when(sid == _NS - 1)
    def _():
        pltpu.sync_copy(acc.at[pl.ds(roff, _RPT_LAST)],
                        out_hbm.at[cid, pl.ds(roff, _RPT_LAST)])


_HI = lax.Precision.HIGHEST


def _dot(a, b):
    return jnp.dot(a, b, precision=_HI, preferred_element_type=_f32)


def _write_h2(h2_ref, hw, dinv):
    h2_ref[:_N] = hw * dinv[:, None]
    h2_ref[_N:] = jnp.zeros((_NPADROWS, _D), _f32)


def _pre_body(degp_ref, x_ref, w_ref, dinv_ref, hw_ref, h2_ref):
    degp = degp_ref[...]
    deg = degp[0] + degp[1] + 1.0
    dinv = lax.rsqrt(deg)
    dinv_ref[...] = dinv
    hw = _dot(x_ref[...], w_ref[...])
    hw_ref[...] = hw
    _write_h2(h2_ref, hw, dinv)


_pre_call = pl.pallas_call(
    _pre_body,
    out_shape=[
        jax.ShapeDtypeStruct((_N,), _f32),
        jax.ShapeDtypeStruct((_N, _D), _f32),
        jax.ShapeDtypeStruct((_NPAD, _D), _f32),
    ],
)


def _proj_body(h_ref, w_ref, dinv_ref, hw_ref, h2_ref):
    hw = _dot(h_ref[...], w_ref[...])
    hw_ref[...] = hw
    _write_h2(h2_ref, hw, dinv_ref[...])


_proj_call = pl.pallas_call(
    _proj_body,
    out_shape=[
        jax.ShapeDtypeStruct((_N, _D), _f32),
        jax.ShapeDtypeStruct((_NPAD, _D), _f32),
    ],
)


def _conv_body(accp_ref, hw_ref, dinv_ref, bias_ref, conv_ref):
    dinv = dinv_ref[...]
    acc = accp_ref[0] + accp_ref[1]
    conv_ref[...] = (acc * dinv[:, None]
                     + hw_ref[...] * (dinv * dinv)[:, None]
                     + bias_ref[...][None, :])


_conv_call = pl.pallas_call(
    _conv_body,
    out_shape=jax.ShapeDtypeStruct((_N, _D), _f32),
)


def _graph_norm(conv_ref, hprev_ref, gw_ref, gb_ref, ga_ref, batch_ref):
    conv = conv_ref[...]
    gids = lax.broadcasted_iota(jnp.int32, (_NG, _N), 0)
    P = (gids == batch_ref[...][None, :]).astype(_f32)
    cnt = jnp.maximum(jnp.sum(P, axis=1), 1.0)
    mean = _dot(P, conv) / cnt[:, None]
    mean_s = lax.dot_general(P, mean, (((0,), (0,)), ((), ())),
                             precision=_HI, preferred_element_type=_f32)
    out = conv - mean_s * ga_ref[...][None, :]
    var = _dot(P, out * out) / cnt[:, None]
    rstd = lax.rsqrt(var + _EPS)
    rstd_s = lax.dot_general(P, rstd, (((0,), (0,)), ((), ())),
                             precision=_HI, preferred_element_type=_f32)
    outn = out * rstd_s * gw_ref[...][None, :] + gb_ref[...][None, :]
    h_new = jnp.maximum(outn, 0.0) + hprev_ref[...]
    return h_new, P, cnt


def _gn_body(conv_ref, hprev_ref, gw_ref, gb_ref, ga_ref, batch_ref, h_ref):
    h_new, _, _ = _graph_norm(conv_ref, hprev_ref, gw_ref, gb_ref, ga_ref,
                              batch_ref)
    h_ref[...] = h_new


_gn_call = pl.pallas_call(
    _gn_body,
    out_shape=jax.ShapeDtypeStruct((_N, _D), _f32),
)


def _gnpool_body(conv_ref, hprev_ref, gw_ref, gb_ref, ga_ref, batch_ref,
                 g_ref):
    h_new, P, cnt = _graph_norm(conv_ref, hprev_ref, gw_ref, gb_ref, ga_ref,
                                batch_ref)
    g_ref[...] = _dot(P, h_new) / cnt[:, None]


_gnpool_call = pl.pallas_call(
    _gnpool_body,
    out_shape=jax.ShapeDtypeStruct((_NG, _D), _f32),
)


def kernel(x, edge_index, batch, W, b, gn_w, gn_b, gn_a):
    src = jnp.concatenate([edge_index[0], edge_index[1]])
    dst = jnp.concatenate([edge_index[1], edge_index[0]])
    keys = src * _N + dst
    sk = lax.sort(keys, is_stable=False)
    first = jnp.concatenate([jnp.ones((1,), bool), sk[1:] != sk[:-1]])
    a = sk // _N
    bb = sk % _N
    e = jnp.arange(_E2, dtype=jnp.int32)
    gidx = jnp.where(first, bb, _N + (e & (_NPADROWS - 1)))
    pad_e = jnp.arange(_E2, _EPAD, dtype=jnp.int32)
    gidx = jnp.concatenate([gidx, _N + (pad_e & (_NPADROWS - 1))])
    sidx = jnp.concatenate([a, pad_e % _N]).astype(jnp.int32)
    ew = jnp.concatenate([first.astype(_f32), jnp.zeros((_EPAD - _E2,), _f32)])
    gidx = gidx.astype(jnp.int32).reshape(_NW, _CPW, _CH)
    sidx = sidx.reshape(_NW, _CPW, _CH)
    ew = ew.reshape(_NW, _CPW, _CH)

    zeros1 = jnp.zeros((_N,), _f32)
    zeros2 = jnp.zeros((_N, _D), _f32)

    degp = _deg_kernel(sidx, ew, zeros1)
    dinv, hw, h2 = _pre_call(degp, x, W[0])

    h = x
    for i in range(_L):
        accp = _agg_kernel(h2, gidx, sidx, zeros2)
        conv = _conv_call(accp, hw, dinv, b[i])
        if i < _L - 1:
            h = _gn_call(conv, h, gn_w[i], gn_b[i], gn_a[i], batch)
            hw, h2 = _proj_call(h, W[i + 1], dinv)
        else:
            g = _gnpool_call(conv, h, gn_w[i], gn_b[i], gn_a[i], batch)
    return g


# speedup vs baseline: 1.1624x; 1.1624x over previous
import functools

import jax
import jax.numpy as jnp
from jax import lax
from jax.experimental import pallas as pl
from jax.experimental.pallas import tpu as pltpu
from jax.experimental.pallas import tpu_sc as plsc

_N = 10000
_E = 320000
_D = 128
_DH = _D // 2
_NG = 64
_L = 3
_EPS = 1e-5

_NC, _NS = 2, 16
_NW = _NC * _NS
_CH = 128
_IB = 32
_E2 = 2 * _E
_EPW = -(-_E2 // (_NW * _CH * _IB)) * (_CH * _IB)
_CPW = _EPW // _CH
_NBLK = _CPW // _IB
_EPAD = _NW * _EPW
_NPADROWS = 16
_NPAD = _N + _NPADROWS
_RPT = 640
_RPT_LAST = _N - _RPT * (_NS - 1)


_mesh = plsc.VectorSubcoreMesh(core_axis_name="c", subcore_axis_name="s",
                               num_cores=_NC, num_subcores=_NS)
_f32 = jnp.float32


@functools.partial(
    pl.kernel,
    mesh=_mesh,
    out_type=jax.ShapeDtypeStruct((_NC, _N), _f32),
    scratch_types=[
        pltpu.VMEM((_CPW, _CH), jnp.int32),
        pltpu.VMEM((_CPW, _CH), _f32),
        pltpu.VMEM_SHARED((_N,), _f32),
    ],
)
def _deg_kernel(sidx_hbm, ew_hbm, zeros1_hbm, out_hbm, idx_v, ew_v, dacc):
    cid = lax.axis_index("c")
    sid = lax.axis_index("s")
    wid = cid * _NS + sid
    pltpu.sync_copy(sidx_hbm.at[wid], idx_v)
    pltpu.sync_copy(ew_hbm.at[wid], ew_v)

    @pl.when(sid == 0)
    def _():
        pltpu.sync_copy(zeros1_hbm, dacc)

    plsc.subcore_barrier()

    def body(j, c):
        pltpu.sync_copy(ew_v.at[j], dacc.at[idx_v.at[j]], add=True)
        return c

    lax.fori_loop(0, _CPW, body, 0)
    plsc.subcore_barrier()

    @pl.when(sid == 0)
    def _():
        pltpu.sync_copy(dacc, out_hbm.at[cid])


@functools.partial(
    pl.kernel,
    mesh=_mesh,
    out_type=jax.ShapeDtypeStruct((_NC, _N, _D), _f32),
    scratch_types=[
        pltpu.VMEM((_IB, _CH), jnp.int32),
        pltpu.VMEM((_IB, _CH), jnp.int32),
        pltpu.VMEM((_CH, _D), _f32),
        pltpu.VMEM((_CH, _D), _f32),
        pltpu.VMEM_SHARED((_N, _D), _f32),
        pltpu.SemaphoreType.DMA,
        pltpu.SemaphoreType.DMA,
        pltpu.SemaphoreType.DMA,
        pltpu.SemaphoreType.DMA,
    ],
)
def _agg_kernel(h2_hbm, gidx_hbm, sidx_hbm, zeros2_hbm, out_hbm,
                gidx_v, sidx_v, rows0, rows1, acc, g0, g1, s0, s1):
    cid = lax.axis_index("c")
    sid = lax.axis_index("s")
    wid = cid * _NS + sid
    roff = pl.multiple_of(sid * _RPT, 8)

    @pl.when(sid < _NS - 1)
    def _():
        pltpu.sync_copy(zeros2_hbm.at[pl.ds(roff, _RPT)],
                        acc.at[pl.ds(roff, _RPT)])

    @pl.when(sid == _NS - 1)
    def _():
        pltpu.sync_copy(zeros2_hbm.at[pl.ds(roff, _RPT_LAST)],
                        acc.at[pl.ds(roff, _RPT_LAST)])

    plsc.subcore_barrier()

    def blk_body(blk, c):
        boff = pl.multiple_of(blk * _IB, _IB)
        pltpu.sync_copy(gidx_hbm.at[wid, pl.ds(boff, _IB)], gidx_v)
        pltpu.sync_copy(sidx_hbm.at[wid, pl.ds(boff, _IB)], sidx_v)
        pltpu.async_copy(h2_hbm.at[gidx_v.at[0]], rows0, g0)

        def pair(p, c2):
            j = 2 * p

            @pl.when(p > 0)
            def _():
                pltpu.make_async_copy(rows1, acc.at[sidx_v.at[j - 1]],
                                      s1).wait()

            g1d = pltpu.async_copy(h2_hbm.at[gidx_v.at[j + 1]], rows1, g1)
            pltpu.make_async_copy(h2_hbm.at[gidx_v.at[j]], rows0, g0).wait()
            s0d = pltpu.async_copy(rows0, acc.at[sidx_v.at[j]], s0, add=True)
            s0d.wait()

            @pl.when(p < _IB // 2 - 1)
            def _():
                pltpu.async_copy(h2_hbm.at[gidx_v.at[j + 2]], rows0, g0)

            g1d.wait()
            pltpu.async_copy(rows1, acc.at[sidx_v.at[j + 1]], s1, add=True)
            return c2

        c = lax.fori_loop(0, _IB // 2, pair, c, unroll=4)
        pltpu.make_async_copy(rows1, acc.at[sidx_v.at[_IB - 1]], s1).wait()
        return c

    lax.fori_loop(0, _NBLK, blk_body, 0)
    plsc.subcore_barrier()

    @pl.when(sid < _NS - 1)
    def _():
        pltpu.sync_copy(acc.at[pl.ds(roff, _RPT)],
                        out_hbm.at[cid, pl.ds(roff, _RPT)])

    @pl.when(sid == _NS - 1)
    def _():
        pltpu.sync_copy(acc.at[pl.ds(roff, _RPT_LAST)],
                        out_hbm.at[cid, pl.ds(roff, _RPT_LAST)])


_HI = lax.Precision.HIGHEST


def _dot(a, b):
    return jnp.dot(a, b, precision=_HI, preferred_element_type=_f32)


def _write_h2(h2_ref, hw, dinv):
    h2_ref[:_N] = hw * dinv[:, None]
    h2_ref[_N:] = jnp.zeros((_NPADROWS, _D), _f32)


def _pre_body(degp_ref, x_ref, w_ref, dinv_ref, hw_ref, h2_ref):
    degp = degp_ref[...]
    deg = degp[0] + degp[1] + 1.0
    dinv = lax.rsqrt(deg)
    dinv_ref[...] = dinv
    hw = _dot(x_ref[...], w_ref[...])
    hw_ref[...] = hw
    _write_h2(h2_ref, hw, dinv)


_pre_call = pl.pallas_call(
    _pre_body,
    out_shape=[
        jax.ShapeDtypeStruct((_N,), _f32),
        jax.ShapeDtypeStruct((_N, _D), _f32),
        jax.ShapeDtypeStruct((_NPAD, _D), _f32),
    ],
)


def _proj_body(h_ref, w_ref, dinv_ref, hw_ref, h2_ref):
    hw = _dot(h_ref[...], w_ref[...])
    hw_ref[...] = hw
    _write_h2(h2_ref, hw, dinv_ref[...])


_proj_call = pl.pallas_call(
    _proj_body,
    out_shape=[
        jax.ShapeDtypeStruct((_N, _D), _f32),
        jax.ShapeDtypeStruct((_NPAD, _D), _f32),
    ],
)


def _conv_body(accp_ref, hw_ref, dinv_ref, bias_ref, conv_ref):
    dinv = dinv_ref[...]
    acc = accp_ref[0] + accp_ref[1]
    conv_ref[...] = (acc * dinv[:, None]
                     + hw_ref[...] * (dinv * dinv)[:, None]
                     + bias_ref[...][None, :])


_conv_call = pl.pallas_call(
    _conv_body,
    out_shape=jax.ShapeDtypeStruct((_N, _D), _f32),
)


def _graph_norm(conv_ref, hprev_ref, gw_ref, gb_ref, ga_ref, batch_ref):
    conv = conv_ref[...]
    gids = lax.broadcasted_iota(jnp.int32, (_NG, _N), 0)
    P = (gids == batch_ref[...][None, :]).astype(_f32)
    cnt = jnp.maximum(jnp.sum(P, axis=1), 1.0)
    mean = _dot(P, conv) / cnt[:, None]
    mean_s = lax.dot_general(P, mean, (((0,), (0,)), ((), ())),
                             precision=_HI, preferred_element_type=_f32)
    out = conv - mean_s * ga_ref[...][None, :]
    var = _dot(P, out * out) / cnt[:, None]
    rstd = lax.rsqrt(var + _EPS)
    rstd_s = lax.dot_general(P, rstd, (((0,), (0,)), ((), ())),
                             precision=_HI, preferred_element_type=_f32)
    outn = out * rstd_s * gw_ref[...][None, :] + gb_ref[...][None, :]
    h_new = jnp.maximum(outn, 0.0) + hprev_ref[...]
    return h_new, P, cnt


def _gn_body(conv_ref, hprev_ref, gw_ref, gb_ref, ga_ref, batch_ref, h_ref):
    h_new, _, _ = _graph_norm(conv_ref, hprev_ref, gw_ref, gb_ref, ga_ref,
                              batch_ref)
    h_ref[...] = h_new


_gn_call = pl.pallas_call(
    _gn_body,
    out_shape=jax.ShapeDtypeStruct((_N, _D), _f32),
)


def _gnpool_body(conv_ref, hprev_ref, gw_ref, gb_ref, ga_ref, batch_ref,
                 g_ref):
    h_new, P, cnt = _graph_norm(conv_ref, hprev_ref, gw_ref, gb_ref, ga_ref,
                                batch_ref)
    g_ref[...] = _dot(P, h_new) / cnt[:, None]


_gnpool_call = pl.pallas_call(
    _gnpool_body,
    out_shape=jax.ShapeDtypeStruct((_NG, _D), _f32),
)


def kernel(x, edge_index, batch, W, b, gn_w, gn_b, gn_a):
    src = jnp.concatenate([edge_index[0], edge_index[1]])
    dst = jnp.concatenate([edge_index[1], edge_index[0]])
    keys = src * _N + dst
    sk = lax.sort(keys, is_stable=False)
    first = jnp.concatenate([jnp.ones((1,), bool), sk[1:] != sk[:-1]])
    a = sk // _N
    bb = sk % _N
    e = jnp.arange(_E2, dtype=jnp.int32)
    gidx = jnp.where(first, bb, _N + (e & (_NPADROWS - 1)))
    pad_e = jnp.arange(_E2, _EPAD, dtype=jnp.int32)
    gidx = jnp.concatenate([gidx, _N + (pad_e & (_NPADROWS - 1))])
    sidx = jnp.concatenate([a, pad_e % _N]).astype(jnp.int32)
    ew = jnp.concatenate([first.astype(_f32), jnp.zeros((_EPAD - _E2,), _f32)])
    gidx = gidx.astype(jnp.int32).reshape(_NW, _CPW, _CH)
    sidx = sidx.reshape(_NW, _CPW, _CH)
    ew = ew.reshape(_NW, _CPW, _CH)

    zeros1 = jnp.zeros((_N,), _f32)
    zeros2 = jnp.zeros((_N, _D), _f32)

    degp = _deg_kernel(sidx, ew, zeros1)
    dinv, hw, h2 = _pre_call(degp, x, W[0])

    h = x
    for i in range(_L):
        accp = _agg_kernel(h2, gidx, sidx, zeros2)
        conv = _conv_call(accp, hw, dinv, b[i])
        if i < _L - 1:
            h = _gn_call(conv, h, gn_w[i], gn_b[i], gn_a[i], batch)
            hw, h2 = _proj_call(h, W[i + 1], dinv)
        else:
            g = _gnpool_call(conv, h, gn_w[i], gn_b[i], gn_a[i], batch)
    return g
